# baseline (speedup 1.0000x reference)
import jax
import jax.numpy as jnp
from jax import lax
from jax.experimental import pallas as pl
from jax.experimental.pallas import tpu as pltpu

_N = 64
_K = 3072
_F = 4096
_MIN = -2**31


def _corr_body(x_ref, corr_ref, c4k_ref):
    x = x_ref[0]
    mean = jnp.mean(x, axis=-1, keepdims=True)
    xc = x - mean
    var = jnp.sum(xc * xc, axis=-1, keepdims=True) / (_F - 1)
    std = jnp.sqrt(var) + 1e-8
    xn = xc / std
    corr = lax.dot_general(xn, xn, (((1,), (1,)), ((), ())),
                           preferred_element_type=jnp.float32) / _F
    corr_ref[0] = corr
    flat = corr
    while flat.shape[0] > 1:
        h = flat.shape[0] // 2
        flat = jnp.concatenate([flat[:h], flat[h:]], axis=1)
    c4k_ref[...] = flat[None]


def _thr_body(c_ref, thr_ref):
    c = c_ref[:, 0, :]
    t = jnp.concatenate(
        [jnp.transpose(c[:, 1024 * i:1024 * (i + 1)]) for i in range(4)], axis=1)
    b = lax.bitcast_convert_type(t, jnp.int32)
    skey = b ^ ((b >> 31) & jnp.int32(0x7FFFFFFF))

    uprefix = jnp.zeros((1, 128), jnp.int32)
    for bit in range(31, -1, -1):
        utrial = uprefix | jnp.int32(_MIN if bit == 31 else 1 << bit)
        m = skey < (utrial ^ jnp.int32(_MIN))
        cnt = jnp.sum(jnp.where(m, 1.0, 0.0), axis=0, keepdims=True)
        cntb = (cnt + pltpu.roll(cnt, 32, 1)
                + pltpu.roll(cnt, 64, 1) + pltpu.roll(cnt, 96, 1))
        uprefix = jnp.where(cntb >= _K, uprefix, utrial)
    skey_thr = uprefix ^ jnp.int32(_MIN)
    bthr = skey_thr ^ ((skey_thr >> 31) & jnp.int32(0x7FFFFFFF))
    thr = lax.bitcast_convert_type(bthr, jnp.float32)
    thr_b = jnp.transpose(thr[:, :32])
    thr_ref[...] = jnp.broadcast_to(thr_b[:, :, None], thr_ref.shape)


def _adj_body(corr_ref, thr_ref, adj_ref):
    c = corr_ref[0]
    t = thr_ref[0, 0, 0]
    row = lax.broadcasted_iota(jnp.int32, (_N, _N), 0)
    col = lax.broadcasted_iota(jnp.int32, (_N, _N), 1)
    adj_ref[0] = jnp.where((c > t) & (row != col), 1.0, 0.0)


def kernel(H):
    B = H.shape[0]
    X = H.reshape(B, _N, _F)
    corr, c4k = pl.pallas_call(
        _corr_body,
        grid=(B,),
        in_specs=[pl.BlockSpec((1, _N, _F), lambda i: (i, 0, 0))],
        out_specs=[
            pl.BlockSpec((1, _N, _N), lambda i: (i, 0, 0)),
            pl.BlockSpec((1, 1, _N * _N), lambda i: (i, 0, 0)),
        ],
        out_shape=[
            jax.ShapeDtypeStruct((B, _N, _N), jnp.float32),
            jax.ShapeDtypeStruct((B, 1, _N * _N), jnp.float32),
        ],
    )(X)
    thr = pl.pallas_call(
        _thr_body,
        out_shape=jax.ShapeDtypeStruct((B, 1, 128), jnp.float32),
        in_specs=[pl.BlockSpec((B, 1, _N * _N), lambda: (0, 0, 0))],
        out_specs=pl.BlockSpec((B, 1, 128), lambda: (0, 0, 0)),
    )(c4k)
    adj = pl.pallas_call(
        _adj_body,
        grid=(B,),
        in_specs=[
            pl.BlockSpec((1, _N, _N), lambda i: (i, 0, 0)),
            pl.BlockSpec((1, 1, 128), lambda i: (i, 0, 0)),
        ],
        out_specs=pl.BlockSpec((1, _N, _N), lambda i: (i, 0, 0)),
        out_shape=jax.ShapeDtypeStruct((B, _N, _N), jnp.float32),
    )(corr, thr)
    return (X, adj)


# speedup vs baseline: 1.2655x; 1.2655x over previous
import jax
import jax.numpy as jnp
from jax import lax
from jax.experimental import pallas as pl
from jax.experimental.pallas import tpu as pltpu

_N = 64
_K = 3072
_F = 4096
_MIN = -2**31


def _corr_body(h_ref, nf_ref, corr_ref, c4k_ref):
    cols = [h_ref[0, pl.Slice(j, 64, 32), :] for j in range(32)]
    x = jnp.concatenate(cols, axis=1)
    nf_ref[0] = x
    mean = jnp.mean(x, axis=-1, keepdims=True)
    xc = x - mean
    var = jnp.sum(xc * xc, axis=-1, keepdims=True) / (_F - 1)
    std = jnp.sqrt(var) + 1e-8
    xn = xc / std
    corr = lax.dot_general(xn, xn, (((1,), (1,)), ((), ())),
                           preferred_element_type=jnp.float32) / _F
    corr_ref[0] = corr
    flat = corr
    while flat.shape[0] > 1:
        hh = flat.shape[0] // 2
        flat = jnp.concatenate([flat[:hh], flat[hh:]], axis=1)
    c4k_ref[...] = flat[None]


def _thr_body(c_ref, thr_ref):
    c = c_ref[:, 0, :]
    t = jnp.concatenate(
        [jnp.transpose(c[:, 1024 * i:1024 * (i + 1)]) for i in range(4)], axis=1)
    b = lax.bitcast_convert_type(t, jnp.int32)
    skey = b ^ ((b >> 31) & jnp.int32(0x7FFFFFFF))

    uprefix = jnp.zeros((1, 128), jnp.int32)
    for bit in range(31, -1, -1):
        utrial = uprefix | jnp.int32(_MIN if bit == 31 else 1 << bit)
        m = skey < (utrial ^ jnp.int32(_MIN))
        cnt = jnp.sum(jnp.where(m, 1.0, 0.0), axis=0, keepdims=True)
        cntb = (cnt + pltpu.roll(cnt, 32, 1)
                + pltpu.roll(cnt, 64, 1) + pltpu.roll(cnt, 96, 1))
        uprefix = jnp.where(cntb >= _K, uprefix, utrial)
    skey_thr = uprefix ^ jnp.int32(_MIN)
    bthr = skey_thr ^ ((skey_thr >> 31) & jnp.int32(0x7FFFFFFF))
    thr = lax.bitcast_convert_type(bthr, jnp.float32)
    thr_b = jnp.transpose(thr[:, :32])
    thr_ref[...] = jnp.broadcast_to(thr_b[:, :, None], thr_ref.shape)


def _adj_body(corr_ref, thr_ref, adj_ref):
    c = corr_ref[0]
    t = thr_ref[0, 0, 0]
    row = lax.broadcasted_iota(jnp.int32, (_N, _N), 0)
    col = lax.broadcasted_iota(jnp.int32, (_N, _N), 1)
    adj_ref[0] = jnp.where((c > t) & (row != col), 1.0, 0.0)


def kernel(H):
    B, S, D = H.shape
    nf, corr, c4k = pl.pallas_call(
        _corr_body,
        grid=(B,),
        in_specs=[pl.BlockSpec((1, S, D), lambda i: (i, 0, 0))],
        out_specs=[
            pl.BlockSpec((1, _N, _F), lambda i: (i, 0, 0)),
            pl.BlockSpec((1, _N, _N), lambda i: (i, 0, 0)),
            pl.BlockSpec((1, 1, _N * _N), lambda i: (i, 0, 0)),
        ],
        out_shape=[
            jax.ShapeDtypeStruct((B, _N, _F), jnp.float32),
            jax.ShapeDtypeStruct((B, _N, _N), jnp.float32),
            jax.ShapeDtypeStruct((B, 1, _N * _N), jnp.float32),
        ],
    )(H)
    thr = pl.pallas_call(
        _thr_body,
        out_shape=jax.ShapeDtypeStruct((B, 1, 128), jnp.float32),
        in_specs=[pl.BlockSpec((B, 1, _N * _N), lambda: (0, 0, 0))],
        out_specs=pl.BlockSpec((B, 1, 128), lambda: (0, 0, 0)),
    )(c4k)
    adj = pl.pallas_call(
        _adj_body,
        grid=(B,),
        in_specs=[
            pl.BlockSpec((1, _N, _N), lambda i: (i, 0, 0)),
            pl.BlockSpec((1, 1, 128), lambda i: (i, 0, 0)),
        ],
        out_specs=pl.BlockSpec((1, _N, _N), lambda i: (i, 0, 0)),
        out_shape=jax.ShapeDtypeStruct((B, _N, _N), jnp.float32),
    )(corr, thr)
    return (nf, adj)


# speedup vs baseline: 1.3894x; 1.0979x over previous
import jax
import jax.numpy as jnp
from jax import lax
from jax.experimental import pallas as pl
from jax.experimental.pallas import tpu as pltpu

_N = 64
_B = 32
_K = 3072
_F = 4096
_MIN = -2**31


def _body(h_ref, nf_ref, adj_ref, corr_scr, c4k_scr, thr_scr):
    i = pl.program_id(0)

    @pl.when(i < _B)
    def _phase_corr():
        cols = [h_ref[0, pl.Slice(j, _N, _B), :] for j in range(32)]
        x = jnp.concatenate(cols, axis=1)
        nf_ref[0] = x
        mean = jnp.mean(x, axis=-1, keepdims=True)
        xc = x - mean
        var = jnp.sum(xc * xc, axis=-1, keepdims=True) / (_F - 1)
        std = jnp.sqrt(var) + 1e-8
        xn = xc / std
        corr = lax.dot_general(xn, xn, (((1,), (1,)), ((), ())),
                               preferred_element_type=jnp.float32) / _F
        corr_scr[i] = corr
        flat = corr
        while flat.shape[0] > 1:
            hh = flat.shape[0] // 2
            flat = jnp.concatenate([flat[:hh], flat[hh:]], axis=1)
        c4k_scr[i] = flat

    @pl.when(i == _B)
    def _phase_select():
        c = c4k_scr[:, 0, :]
        t = jnp.concatenate(
            [jnp.transpose(c[:, 1024 * j:1024 * (j + 1)]) for j in range(4)],
            axis=1)
        b = lax.bitcast_convert_type(t, jnp.int32)
        skey = b ^ ((b >> 31) & jnp.int32(0x7FFFFFFF))
        uprefix = jnp.zeros((1, 128), jnp.int32)
        for bit in range(31, -1, -1):
            utrial = uprefix | jnp.int32(_MIN if bit == 31 else 1 << bit)
            m = skey < (utrial ^ jnp.int32(_MIN))
            cnt = jnp.sum(jnp.where(m, 1.0, 0.0), axis=0, keepdims=True)
            cntb = (cnt + pltpu.roll(cnt, 32, 1)
                    + pltpu.roll(cnt, 64, 1) + pltpu.roll(cnt, 96, 1))
            uprefix = jnp.where(cntb >= _K, uprefix, utrial)
        skey_thr = uprefix ^ jnp.int32(_MIN)
        bthr = skey_thr ^ ((skey_thr >> 31) & jnp.int32(0x7FFFFFFF))
        thr_scr[...] = lax.bitcast_convert_type(bthr, jnp.float32)

    @pl.when(i > _B)
    def _phase_adj():
        bi = i - (_B + 1)
        c = corr_scr[bi]
        lane = lax.broadcasted_iota(jnp.int32, (1, 128), 1)
        trow = thr_scr[...]
        tval = jnp.sum(jnp.where(lane == bi, trow, 0.0))
        row = lax.broadcasted_iota(jnp.int32, (_N, _N), 0)
        col = lax.broadcasted_iota(jnp.int32, (_N, _N), 1)
        adj_ref[0] = jnp.where((c > tval) & (row != col), 1.0, 0.0)


def kernel(H):
    B, S, D = H.shape
    nf, adj = pl.pallas_call(
        _body,
        grid=(2 * B + 1,),
        in_specs=[pl.BlockSpec((1, S, D),
                               lambda i: (jnp.minimum(i, _B - 1), 0, 0))],
        out_specs=[
            pl.BlockSpec((1, _N, _F),
                         lambda i: (jnp.minimum(i, _B - 1), 0, 0)),
            pl.BlockSpec((1, _N, _N),
                         lambda i: (jnp.clip(i - _B - 1, 0, _B - 1), 0, 0)),
        ],
        out_shape=[
            jax.ShapeDtypeStruct((B, _N, _F), jnp.float32),
            jax.ShapeDtypeStruct((B, _N, _N), jnp.float32),
        ],
        scratch_shapes=[
            pltpu.VMEM((_B, _N, _N), jnp.float32),
            pltpu.VMEM((_B, 1, _F), jnp.float32),
            pltpu.VMEM((1, 128), jnp.float32),
        ],
    )(H)
    return (nf, adj)


# speedup vs baseline: 1.6461x; 1.1847x over previous
import jax
import jax.numpy as jnp
from jax import lax
from jax.experimental import pallas as pl
from jax.experimental.pallas import tpu as pltpu

_N = 64
_B = 32
_K = 3072
_F = 4096
_MIN = -2**31


def _body(h_ref, nf_ref, adj_ref, corr_scr, c4k_scr, thr_scr):
    i = pl.program_id(0)

    @pl.when(i < _B)
    def _phase_corr():
        cols = [h_ref[0, pl.Slice(j, _N, _B), :] for j in range(32)]
        x = jnp.concatenate(cols, axis=1)
        nf_ref[0] = x
        mean = jnp.mean(x, axis=-1, keepdims=True)
        xc = x - mean
        var = jnp.sum(xc * xc, axis=-1, keepdims=True) / (_F - 1)
        std = jnp.sqrt(var) + 1e-8
        xn = xc / std
        corr = lax.dot_general(xn, xn, (((1,), (1,)), ((), ())),
                               preferred_element_type=jnp.float32) / _F
        corr_scr[i] = corr
        flat = corr
        while flat.shape[0] > 1:
            hh = flat.shape[0] // 2
            flat = jnp.concatenate([flat[:hh], flat[hh:]], axis=1)
        c4k_scr[i] = flat

    @pl.when(i == _B)
    def _phase_select():
        c = c4k_scr[:, 0, :]
        t = jnp.concatenate(
            [jnp.transpose(c[:, 1024 * j:1024 * (j + 1)]) for j in range(4)],
            axis=1)
        b = lax.bitcast_convert_type(t, jnp.int32)
        skey = b ^ ((b >> 31) & jnp.int32(0x7FFFFFFF))
        uprefix = jnp.zeros((1, 128), jnp.int32)
        for bit in range(31, -1, -1):
            utrial = uprefix | jnp.int32(_MIN if bit == 31 else 1 << bit)
            m = skey < (utrial ^ jnp.int32(_MIN))
            cnt = jnp.sum(jnp.where(m, 1.0, 0.0), axis=0, keepdims=True)
            cntb = (cnt + pltpu.roll(cnt, 32, 1)
                    + pltpu.roll(cnt, 64, 1) + pltpu.roll(cnt, 96, 1))
            uprefix = jnp.where(cntb >= _K, uprefix, utrial)
        skey_thr = uprefix ^ jnp.int32(_MIN)
        bthr = skey_thr ^ ((skey_thr >> 31) & jnp.int32(0x7FFFFFFF))
        thr_scr[...] = lax.bitcast_convert_type(bthr, jnp.float32)

    @pl.when(i > _B)
    def _phase_adj():
        c = corr_scr[...]
        tb = jnp.transpose(thr_scr[:, :_B])
        row = lax.broadcasted_iota(jnp.int32, (_B, _N, _N), 1)
        col = lax.broadcasted_iota(jnp.int32, (_B, _N, _N), 2)
        adj_ref[...] = jnp.where((c > tb[:, :, None]) & (row != col), 1.0, 0.0)


def kernel(H):
    B, S, D = H.shape
    nf, adj = pl.pallas_call(
        _body,
        grid=(B + 2,),
        in_specs=[pl.BlockSpec((1, S, D),
                               lambda i: (jnp.minimum(i, _B - 1), 0, 0))],
        out_specs=[
            pl.BlockSpec((1, _N, _F),
                         lambda i: (jnp.minimum(i, _B - 1), 0, 0)),
            pl.BlockSpec((_B, _N, _N), lambda i: (0, 0, 0)),
        ],
        out_shape=[
            jax.ShapeDtypeStruct((B, _N, _F), jnp.float32),
            jax.ShapeDtypeStruct((B, _N, _N), jnp.float32),
        ],
        scratch_shapes=[
            pltpu.VMEM((_B, _N, _N), jnp.float32),
            pltpu.VMEM((_B, 1, _F), jnp.float32),
            pltpu.VMEM((1, 128), jnp.float32),
        ],
    )(H)
    return (nf, adj)


# speedup vs baseline: 1.7076x; 1.0374x over previous
import jax
import jax.numpy as jnp
from jax import lax
from jax.experimental import pallas as pl
from jax.experimental.pallas import tpu as pltpu

_N = 64
_B = 32
_K = 3072
_F = 4096
_MIN = -2**31


def _i32(x):
    x &= 0xFFFFFFFF
    return x - (1 << 32) if x >= (1 << 31) else x


def _body(h_ref, nf_ref, adj_ref, corr_scr, c4k_scr, thr_scr):
    i = pl.program_id(0)

    @pl.when(i < _B)
    def _phase_corr():
        cols = [h_ref[0, pl.Slice(j, _N, _B), :] for j in range(32)]
        x = jnp.concatenate(cols, axis=1)
        nf_ref[0] = x
        mean = jnp.mean(x, axis=-1, keepdims=True)
        xc = x - mean
        var = jnp.sum(xc * xc, axis=-1, keepdims=True) / (_F - 1)
        std = jnp.sqrt(var) + 1e-8
        xn = xc / std
        corr = lax.dot_general(xn, xn, (((1,), (1,)), ((), ())),
                               preferred_element_type=jnp.float32) / _F
        corr_scr[i] = corr
        flat = corr
        while flat.shape[0] > 1:
            hh = flat.shape[0] // 2
            flat = jnp.concatenate([flat[:hh], flat[hh:]], axis=1)
        c4k_scr[i] = flat

    @pl.when(i == _B)
    def _phase_select():
        c = c4k_scr[:, 0, :]
        t = jnp.concatenate(
            [jnp.transpose(c[:, 1024 * j:1024 * (j + 1)]) for j in range(4)],
            axis=1)
        b = lax.bitcast_convert_type(t, jnp.int32)
        skey = b ^ ((b >> 31) & jnp.int32(0x7FFFFFFF))
        uprefix = jnp.zeros((1, 128), jnp.int32)
        for p in range(30, -1, -2):
            ss = []
            for j in (1, 2, 3):
                utrial = uprefix | jnp.int32(_i32(j << p))
                m = skey < (utrial ^ jnp.int32(_MIN))
                ss.append(jnp.sum(jnp.where(m, 1.0, 0.0), axis=0,
                                  keepdims=True))
            s = jnp.concatenate(ss, axis=0)
            sb = (s + pltpu.roll(s, 32, 1)
                  + pltpu.roll(s, 64, 1) + pltpu.roll(s, 96, 1))
            digit = jnp.sum(jnp.where(sb < _K, 1, 0), axis=0, keepdims=True)
            uprefix = uprefix | (digit << p)
        skey_thr = uprefix ^ jnp.int32(_MIN)
        bthr = skey_thr ^ ((skey_thr >> 31) & jnp.int32(0x7FFFFFFF))
        thr_scr[...] = lax.bitcast_convert_type(bthr, jnp.float32)

    @pl.when(i > _B)
    def _phase_adj():
        c = corr_scr[...]
        tb = jnp.transpose(thr_scr[:, :_B])
        row = lax.broadcasted_iota(jnp.int32, (_B, _N, _N), 1)
        col = lax.broadcasted_iota(jnp.int32, (_B, _N, _N), 2)
        adj_ref[...] = jnp.where((c > tb[:, :, None]) & (row != col), 1.0, 0.0)


def kernel(H):
    B, S, D = H.shape
    nf, adj = pl.pallas_call(
        _body,
        grid=(B + 2,),
        in_specs=[pl.BlockSpec((1, S, D),
                               lambda i: (jnp.minimum(i, _B - 1), 0, 0))],
        out_specs=[
            pl.BlockSpec((1, _N, _F),
                         lambda i: (jnp.minimum(i, _B - 1), 0, 0)),
            pl.BlockSpec((_B, _N, _N), lambda i: (0, 0, 0)),
        ],
        out_shape=[
            jax.ShapeDtypeStruct((B, _N, _F), jnp.float32),
            jax.ShapeDtypeStruct((B, _N, _N), jnp.float32),
        ],
        scratch_shapes=[
            pltpu.VMEM((_B, _N, _N), jnp.float32),
            pltpu.VMEM((_B, 1, _F), jnp.float32),
            pltpu.VMEM((1, 128), jnp.float32),
        ],
    )(H)
    return (nf, adj)
